# speedup vs baseline: 1.0599x; 1.0599x over previous
import functools

import jax
import jax.numpy as jnp
from jax import lax
from jax.experimental import pallas as pl
from jax.experimental.pallas import tpu as pltpu
from jax.experimental.pallas import tpu_sc as plsc

N = 10000
E = 320000
DF = 128
DE = 16
H = 64

INTERPRET = False

NC = 2
NS = 16


def _silu(x):
    return x * jax.nn.sigmoid(x)


def _proj_body(ndata_ref, w_ref, b_ref, out_ref):
    out_ref[...] = (
        jnp.dot(ndata_ref[...], w_ref[...], preferred_element_type=jnp.float32)
        + b_ref[...]
    )


def _node_proj(ndata, wcat, bcat):
    return pl.pallas_call(
        _proj_body,
        out_shape=jax.ShapeDtypeStruct((N, 192), jnp.float32),
        interpret=INTERPRET,
    )(ndata, wcat, bcat)


EB = 8000


def _edge_body(ed_ref, gs_ref, gd_ref, w1e_ref, w2_ref, b2_ref, w3_ref,
               b3_ref, g_ref, bln_ref, m_ref):
    e1 = jnp.dot(ed_ref[...], w1e_ref[...], preferred_element_type=jnp.float32)
    h1 = _silu(gs_ref[...] + gd_ref[...] + e1)
    h2 = _silu(jnp.dot(h1, w2_ref[...], preferred_element_type=jnp.float32)
               + b2_ref[...])
    y = jnp.dot(h2, w3_ref[...], preferred_element_type=jnp.float32) + b3_ref[...]
    mu = jnp.mean(y, axis=-1, keepdims=True)
    var = jnp.mean(y * y, axis=-1, keepdims=True) - mu * mu
    m_ref[...] = (y - mu) * lax.rsqrt(var + 1e-5) * g_ref[...] + bln_ref[...]


def _edge_mlp(edata, g_s, g_d, w1e, w2, b2, w3, b3, g, bln):
    grid = (E // EB,)
    blk = lambda w: pl.BlockSpec((EB, w), lambda i: (i, 0))
    full = lambda a, b: pl.BlockSpec((a, b), lambda i: (0, 0))
    return pl.pallas_call(
        _edge_body,
        grid=grid,
        in_specs=[
            blk(DE), blk(H), blk(H),
            full(DE, H), full(H, H), full(1, H), full(H, DF), full(1, DF),
            full(1, DF), full(1, DF),
        ],
        out_specs=blk(DF),
        out_shape=jax.ShapeDtypeStruct((E, DF), jnp.float32),
        interpret=INTERPRET,
    )(edata, g_s, g_d, w1e, w2, b2, w3, b3, g, bln)


def _node_body(ss_ref, sd_ref, cs_ref, cd_ref, nc_ref, wa_ref, wb_ref,
               w2_ref, b2_ref, w3_ref, b3_ref, g_ref, bln_ref, out_ref):
    cs = jnp.max(cs_ref[...], axis=-1, keepdims=True)
    cd = jnp.max(cd_ref[...], axis=-1, keepdims=True)
    cs = jnp.maximum(cs, 1.0)
    cd = jnp.maximum(cd, 1.0)
    pa = jnp.dot(ss_ref[...], wa_ref[...], preferred_element_type=jnp.float32) / cs
    pb = jnp.dot(sd_ref[...], wb_ref[...], preferred_element_type=jnp.float32) / cd
    h1 = _silu(pa + pb + nc_ref[...])
    h2 = _silu(jnp.dot(h1, w2_ref[...], preferred_element_type=jnp.float32)
               + b2_ref[...])
    y = jnp.dot(h2, w3_ref[...], preferred_element_type=jnp.float32) + b3_ref[...]
    mu = jnp.mean(y, axis=-1, keepdims=True)
    var = jnp.mean(y * y, axis=-1, keepdims=True) - mu * mu
    out_ref[...] = (y - mu) * lax.rsqrt(var + 1e-5) * g_ref[...] + bln_ref[...]


def _node_mlp(sum_s, sum_d, cnt_s, cnt_d, n_c, wa, wb, w2, b2, w3, b3, g, bln):
    return pl.pallas_call(
        _node_body,
        out_shape=jax.ShapeDtypeStruct((N, DF), jnp.float32),
        interpret=INTERPRET,
    )(sum_s, sum_d, cnt_s, cnt_d, n_c, wa, wb, w2, b2, w3, b3, g, bln)


def _gather_proj(p_s, p_d, src, dst):
    return p_s[src], p_d[dst]


def _scatter_sums(m, src, dst):
    sum_s = jax.ops.segment_sum(m, src, num_segments=N)
    sum_d = jax.ops.segment_sum(m, dst, num_segments=N)
    ones = jnp.ones((E, 16), jnp.float32)
    cnt_s = jax.ops.segment_sum(ones, src, num_segments=N)
    cnt_d = jax.ops.segment_sum(ones, dst, num_segments=N)
    return sum_s, cnt_s, sum_d, cnt_d


def kernel(graph, ndata, edata, params):
    src = graph[0].astype(jnp.int32)
    dst = graph[1].astype(jnp.int32)

    eW1, eW2, eW3 = params['edge_Ws']
    eb1, eb2, eb3 = params['edge_bs']
    nW1, nW2, nW3 = params['node_Ws']
    nb1, nb2, nb3 = params['node_bs']

    w1e = eW1[:DE]
    w1s = eW1[DE:DE + DF]
    w1d = eW1[DE + DF:]
    nwa = nW1[:DF]
    nwb = nW1[DF:2 * DF]
    nwc = nW1[2 * DF:]

    wcat = jnp.concatenate([w1s, w1d, nwc], axis=1)
    bcat = jnp.concatenate(
        [eb1, jnp.zeros((H,), jnp.float32), nb1])[None, :]

    proj = _node_proj(ndata, wcat, bcat)
    p_s, p_d, n_c = proj[:, :H], proj[:, H:2 * H], proj[:, 2 * H:]

    g_s, g_d = _gather_proj(p_s, p_d, src, dst)

    m = _edge_mlp(edata, g_s, g_d, w1e, eW2, eb2[None, :], eW3, eb3[None, :],
                  params['edge_ln_g'][None, :], params['edge_ln_b'][None, :])

    sum_s, cnt_s, sum_d, cnt_d = _scatter_sums(m, src, dst)

    node_out = _node_mlp(sum_s, sum_d, cnt_s, cnt_d, n_c, nwa, nwb,
                         nW2, nb2[None, :], nW3, nb3[None, :],
                         params['node_ln_g'][None, :],
                         params['node_ln_b'][None, :])
    return node_out, m


# speedup vs baseline: 1.2977x; 1.2244x over previous
import functools

import jax
import jax.numpy as jnp
from jax import lax
from jax.experimental import pallas as pl
from jax.experimental.pallas import tpu as pltpu
from jax.experimental.pallas import tpu_sc as plsc

N = 10000
E = 320000
DF = 128
DE = 16
H = 64

INTERPRET = False

NC = 2
NS = 16


def _silu(x):
    return x * jax.nn.sigmoid(x)


def _proj_body(ndata_ref, w_ref, b_ref, t_ref, nc_ref):
    y = (jnp.dot(ndata_ref[...], w_ref[...], preferred_element_type=jnp.float32)
         + b_ref[...])
    t_ref[...] = y[:, :2 * H]
    nc_ref[...] = y[:, 2 * H:]


def _node_proj(ndata, wcat, bcat):
    return pl.pallas_call(
        _proj_body,
        out_shape=(jax.ShapeDtypeStruct((N, 2 * H), jnp.float32),
                   jax.ShapeDtypeStruct((N, H), jnp.float32)),
        interpret=INTERPRET,
    )(ndata, wcat, bcat)


EB = 8000


def _edge_body(ed_ref, gs_ref, gd_ref, w1e_ref, w2_ref, b2_ref, w3_ref,
               b3_ref, g_ref, bln_ref, m_ref):
    e1 = jnp.dot(ed_ref[...], w1e_ref[...], preferred_element_type=jnp.float32)
    h1 = _silu(gs_ref[:, :H] + gd_ref[:, H:] + e1)
    h2 = _silu(jnp.dot(h1, w2_ref[...], preferred_element_type=jnp.float32)
               + b2_ref[...])
    y = jnp.dot(h2, w3_ref[...], preferred_element_type=jnp.float32) + b3_ref[...]
    mu = jnp.mean(y, axis=-1, keepdims=True)
    var = jnp.mean(y * y, axis=-1, keepdims=True) - mu * mu
    m_ref[...] = (y - mu) * lax.rsqrt(var + 1e-5) * g_ref[...] + bln_ref[...]


def _edge_mlp(edata, g_s, g_d, w1e, w2, b2, w3, b3, g, bln):
    grid = (E // EB,)
    blk = lambda w: pl.BlockSpec((EB, w), lambda i: (i, 0))
    full = lambda a, b: pl.BlockSpec((a, b), lambda i: (0, 0))
    return pl.pallas_call(
        _edge_body,
        grid=grid,
        in_specs=[
            blk(DE), blk(2 * H), blk(2 * H),
            full(DE, H), full(H, H), full(1, H), full(H, DF), full(1, DF),
            full(1, DF), full(1, DF),
        ],
        out_specs=blk(DF),
        out_shape=jax.ShapeDtypeStruct((E, DF), jnp.float32),
        interpret=INTERPRET,
    )(edata, g_s, g_d, w1e, w2, b2, w3, b3, g, bln)


def _node_body(ss_ref, sd_ref, cs_ref, cd_ref, nc_ref, wa_ref, wb_ref,
               w2_ref, b2_ref, w3_ref, b3_ref, g_ref, bln_ref, out_ref):
    cs = jnp.max(cs_ref[...], axis=-1, keepdims=True)
    cd = jnp.max(cd_ref[...], axis=-1, keepdims=True)
    cs = jnp.maximum(cs, 1.0)
    cd = jnp.maximum(cd, 1.0)
    pa = jnp.dot(ss_ref[...], wa_ref[...], preferred_element_type=jnp.float32) / cs
    pb = jnp.dot(sd_ref[...], wb_ref[...], preferred_element_type=jnp.float32) / cd
    h1 = _silu(pa + pb + nc_ref[...])
    h2 = _silu(jnp.dot(h1, w2_ref[...], preferred_element_type=jnp.float32)
               + b2_ref[...])
    y = jnp.dot(h2, w3_ref[...], preferred_element_type=jnp.float32) + b3_ref[...]
    mu = jnp.mean(y, axis=-1, keepdims=True)
    var = jnp.mean(y * y, axis=-1, keepdims=True) - mu * mu
    out_ref[...] = (y - mu) * lax.rsqrt(var + 1e-5) * g_ref[...] + bln_ref[...]


def _node_mlp(sum_s, sum_d, cnt_s, cnt_d, n_c, wa, wb, w2, b2, w3, b3, g, bln):
    return pl.pallas_call(
        _node_body,
        out_shape=jax.ShapeDtypeStruct((N, DF), jnp.float32),
        interpret=INTERPRET,
    )(sum_s, sum_d, cnt_s, cnt_d, n_c, wa, wb, w2, b2, w3, b3, g, bln)


EPW = E // (NC * NS)
KB = 400


def _sc_gather_body(src_hbm, dst_hbm, t_hbm, gs_hbm, gd_hbm,
                    idxs_v, idxd_v, rows_v, rowd_v, sem, sem2):
    wid = lax.axis_index("s") * NC + lax.axis_index("c")
    base = wid * EPW

    def body(i, carry):
        off = base + i * KB
        pltpu.sync_copy(src_hbm.at[pl.ds(off, KB)], idxs_v)
        pltpu.sync_copy(dst_hbm.at[pl.ds(off, KB)], idxd_v)
        cs = pltpu.async_copy(t_hbm.at[idxs_v], rows_v, sem)
        cd = pltpu.async_copy(t_hbm.at[idxd_v], rowd_v, sem2)
        cs.wait()
        pltpu.sync_copy(rows_v, gs_hbm.at[pl.ds(off, KB)])
        cd.wait()
        pltpu.sync_copy(rowd_v, gd_hbm.at[pl.ds(off, KB)])
        return carry

    lax.fori_loop(0, EPW // KB, body, 0)


def _gather_proj(t, src, dst):
    f = functools.partial(
        pl.kernel,
        out_type=(jax.ShapeDtypeStruct((E, 2 * H), jnp.float32),
                  jax.ShapeDtypeStruct((E, 2 * H), jnp.float32)),
        mesh=plsc.VectorSubcoreMesh(core_axis_name="c", subcore_axis_name="s"),
        scratch_types=[
            pltpu.VMEM((KB,), jnp.int32),
            pltpu.VMEM((KB,), jnp.int32),
            pltpu.VMEM((KB, 2 * H), jnp.float32),
            pltpu.VMEM((KB, 2 * H), jnp.float32),
            pltpu.SemaphoreType.DMA,
            pltpu.SemaphoreType.DMA,
        ],
    )(_sc_gather_body)
    return f(src, dst, t)


def _scatter_sums(m, src, dst):
    sum_s = jax.ops.segment_sum(m, src, num_segments=N)
    sum_d = jax.ops.segment_sum(m, dst, num_segments=N)
    ones = jnp.ones((E, 16), jnp.float32)
    cnt_s = jax.ops.segment_sum(ones, src, num_segments=N)
    cnt_d = jax.ops.segment_sum(ones, dst, num_segments=N)
    return sum_s, cnt_s, sum_d, cnt_d


def kernel(graph, ndata, edata, params):
    src = graph[0].astype(jnp.int32)
    dst = graph[1].astype(jnp.int32)

    eW1, eW2, eW3 = params['edge_Ws']
    eb1, eb2, eb3 = params['edge_bs']
    nW1, nW2, nW3 = params['node_Ws']
    nb1, nb2, nb3 = params['node_bs']

    w1e = eW1[:DE]
    w1s = eW1[DE:DE + DF]
    w1d = eW1[DE + DF:]
    nwa = nW1[:DF]
    nwb = nW1[DF:2 * DF]
    nwc = nW1[2 * DF:]

    wcat = jnp.concatenate([w1s, w1d, nwc], axis=1)
    bcat = jnp.concatenate(
        [eb1, jnp.zeros((H,), jnp.float32), nb1])[None, :]

    t, n_c = _node_proj(ndata, wcat, bcat)

    g_s, g_d = _gather_proj(t, src, dst)

    m = _edge_mlp(edata, g_s, g_d, w1e, eW2, eb2[None, :], eW3, eb3[None, :],
                  params['edge_ln_g'][None, :], params['edge_ln_b'][None, :])

    sum_s, cnt_s, sum_d, cnt_d = _scatter_sums(m, src, dst)

    node_out = _node_mlp(sum_s, sum_d, cnt_s, cnt_d, n_c, nwa, nwb,
                         nW2, nb2[None, :], nW3, nb3[None, :],
                         params['node_ln_g'][None, :],
                         params['node_ln_b'][None, :])
    return node_out, m
